# baseline (speedup 1.0000x reference)
import functools

import jax
import jax.numpy as jnp
from jax import lax
from jax.experimental import pallas as pl
from jax.experimental.pallas import tpu as pltpu
from jax.experimental.pallas import tpu_sc as plsc

_INFO = plsc.get_sparse_core_info()
_NC, _NS = _INFO.num_cores, _INFO.num_subcores
_NW = _NC * _NS

_CHUNK = 8
_NBUF = 2


@functools.partial(jax.jit, static_argnums=(2, 3, 4))
def _gather_rows(idx, weights, n_tokens, d, n_chunks):
    mesh = plsc.VectorSubcoreMesh(core_axis_name="c", subcore_axis_name="s")

    @functools.partial(
        pl.kernel,
        mesh=mesh,
        out_type=jax.ShapeDtypeStruct((n_tokens, d), jnp.float32),
        scratch_types=[
            pltpu.VMEM((n_chunks, _CHUNK), jnp.int32),
            pltpu.VMEM((_NBUF, _CHUNK, d), jnp.float32),
            pltpu.VMEM((_CHUNK, d), jnp.float32),
        ] + [pltpu.SemaphoreType.DMA] * 5,
    )
    def body(idx_hbm, table_hbm, out_hbm, idx_v, rows_v, junk_v, *sems):
        gsems, osems = sems[:2], sems[2:]
        wid = lax.axis_index("s") * _NC + lax.axis_index("c")
        base = wid * (n_chunks * _CHUNK)

        pltpu.sync_copy(idx_hbm.at[wid], idx_v)

        def gather_start(c, b):
            pltpu.async_copy(table_hbm.at[idx_v.at[c]], rows_v.at[b], gsems[b])

        def gather_wait(b):
            pltpu.make_async_copy(
                table_hbm.at[idx_v.at[0]], rows_v.at[b], gsems[b]
            ).wait()

        def scatter_start(b, c):
            pltpu.async_copy(
                junk_v, out_hbm.at[pl.ds(base + c * _CHUNK, _CHUNK)], osems[b]
            )

        def scatter_wait(b):
            pltpu.make_async_copy(
                junk_v, out_hbm.at[pl.ds(base, _CHUNK)], osems[b]
            ).wait()

        gather_start(0, 0)
        gather_start(1, 1)
        scatter_start(0, 0)
        scatter_start(1, 1)
        scatter_start(2, 2)

        def group(g, carry):
            for j in range(2):
                c = g * 2 + 2 + j
                b = j
                gather_wait(b)

                @pl.when(c < n_chunks)
                def _g():
                    gather_start(c, b)
            for j in range(3):
                c = g * 3 + 3 + j
                b = j
                pc = g * 3 + j

                @pl.when((g == 0) | (pc < n_chunks))
                def _w():
                    scatter_wait(b)

                @pl.when(c < n_chunks)
                def _s():
                    scatter_start(b, c)
            return carry

        lax.fori_loop(0, (n_chunks - 2) // 2, group, 0)
        gather_wait(0)
        gather_wait(1)

    return body(idx, weights)


def kernel(x, weights):
    bt, s = x.shape
    v, d = weights.shape
    n = bt * s
    per_w = n // _NW
    n_chunks = per_w // _CHUNK
    idx = x.reshape(_NW, n_chunks, _CHUNK).astype(jnp.int32)
    out = _gather_rows(idx, weights, n, d, n_chunks)
    return out.reshape(bt, s, d)


# speedup vs baseline: 1.0500x; 1.0500x over previous
import functools

import jax
import jax.numpy as jnp
from jax import lax
from jax.experimental import pallas as pl
from jax.experimental.pallas import tpu as pltpu
from jax.experimental.pallas import tpu_sc as plsc

_INFO = plsc.get_sparse_core_info()
_NC, _NS = _INFO.num_cores, _INFO.num_subcores
_NW = _NC * _NS

_CA = 16
_CB = 8
_PAIR = _CA + _CB


@functools.partial(jax.jit, static_argnums=(2, 3, 4))
def _gather_rows(idx, weights, n_tokens, d, per_w):
    mesh = plsc.VectorSubcoreMesh(core_axis_name="c", subcore_axis_name="s")
    n_pairs = per_w // _PAIR
    tail = per_w - n_pairs * _PAIR
    assert tail in (0, _CA)

    @functools.partial(
        pl.kernel,
        mesh=mesh,
        out_type=jax.ShapeDtypeStruct((n_tokens, d), jnp.float32),
        scratch_types=[
            pltpu.VMEM((per_w,), jnp.int32),
            pltpu.VMEM((_CA, d), jnp.float32),
            pltpu.VMEM((_CB, d), jnp.float32),
            pltpu.SemaphoreType.DMA,
            pltpu.SemaphoreType.DMA,
        ],
    )
    def body(idx_hbm, table_hbm, out_hbm, idx_v, rows_a, rows_b, sem_a, sem_b):
        wid = lax.axis_index("s") * _NC + lax.axis_index("c")
        base = wid * per_w

        pltpu.sync_copy(idx_hbm.at[wid], idx_v)

        def ga_start(off):
            pltpu.async_copy(
                table_hbm.at[idx_v.at[pl.ds(off, _CA)]], rows_a, sem_a
            )

        def gb_start(off):
            pltpu.async_copy(
                table_hbm.at[idx_v.at[pl.ds(off, _CB)]], rows_b, sem_b
            )

        def ga_wait():
            pltpu.make_async_copy(
                table_hbm.at[idx_v.at[pl.ds(0, _CA)]], rows_a, sem_a
            ).wait()

        def gb_wait():
            pltpu.make_async_copy(
                table_hbm.at[idx_v.at[pl.ds(0, _CB)]], rows_b, sem_b
            ).wait()

        ga_start(0)
        gb_start(_CA)

        def group(g, carry):
            off = g * _PAIR
            ga_wait()
            pltpu.sync_copy(rows_a, out_hbm.at[pl.ds(base + off, _CA)])

            @pl.when(g + 1 < n_pairs + (1 if tail else 0))
            def _():
                ga_start(off + _PAIR)
            gb_wait()
            pltpu.sync_copy(rows_b, out_hbm.at[pl.ds(base + off + _CA, _CB)])

            @pl.when(g + 1 < n_pairs)
            def _():
                gb_start(off + _PAIR + _CA)
            return carry

        lax.fori_loop(0, n_pairs, group, 0)
        if tail:
            ga_wait()
            pltpu.sync_copy(
                rows_a, out_hbm.at[pl.ds(base + n_pairs * _PAIR, _CA)]
            )

    return body(idx, weights)


def kernel(x, weights):
    bt, s = x.shape
    v, d = weights.shape
    n = bt * s
    per_w = n // _NW
    idx = x.reshape(_NW, per_w).astype(jnp.int32)
    out = _gather_rows(idx, weights, n, d, per_w)
    return out.reshape(bt, s, d)
